# Initial kernel scaffold; baseline (speedup 1.0000x reference)
import jax
import jax.numpy as jnp
from jax.experimental import pallas as pl


def kernel(encoder_out, align_phone, pitch, beats, W_pitch, b_pitch, W_beats, b_beats, W_pos, b_pos):
    raise NotImplementedError("write your pallas kernel here")


# speedup vs baseline: 2.5641x; 2.5641x over previous
import functools

import numpy as np
import jax
import jax.numpy as jnp
from jax import lax
from jax.experimental import pallas as pl
from jax.experimental.pallas import tpu as pltpu
from jax.experimental.pallas import tpu_sc as plsc

B, P, F, H = 16, 512, 4096, 256
NC, NS, L = 2, 16, 16
NW = NC * NS
FB = F // NW
FBLK = 512


def _pe_np():
    pos = np.arange(F, dtype=np.float32)[:, None]
    div = np.exp(np.arange(0, H, 2).astype(np.float32) * (-np.log(10000.0) / H))
    pe = np.zeros((F, H), dtype=np.float32)
    pe[:, 0::2] = np.sin(pos * div)
    pe[:, 1::2] = np.cos(pos * div)
    return pe


_PE = _pe_np()


def _tc_prep_body(pe_ref, wpos_ref, bsum_ref, align_ref, posd_ref, gidx_ref):
    posd_ref[...] = (
        jnp.dot(pe_ref[...], wpos_ref[...], preferred_element_type=jnp.float32)
        + bsum_ref[...]
    )

    @pl.when(pl.program_id(0) == 0)
    def _():
        a = align_ref[...]
        prev = jnp.concatenate([jnp.zeros((B, 1), a.dtype), a[:, :-1]], axis=1)
        x = (a != prev).astype(jnp.int32)
        k = 1
        while k < F:
            shifted = jnp.concatenate(
                [jnp.zeros((B, k), jnp.int32), x[:, : F - k]], axis=1
            )
            x = x + shifted
            k *= 2
        idx = jnp.clip(x, 0, P - 1)
        b_iota = lax.broadcasted_iota(jnp.int32, (B, F), 0)
        gidx_ref[...] = idx + b_iota * P


def _tc_prep(pe, w_pos, bsum, align_phone):
    return pl.pallas_call(
        _tc_prep_body,
        grid=(F // FBLK,),
        in_specs=[
            pl.BlockSpec((FBLK, H), lambda i: (i, 0)),
            pl.BlockSpec((H, H), lambda i: (0, 0)),
            pl.BlockSpec((1, H), lambda i: (0, 0)),
            pl.BlockSpec((B, F), lambda i: (0, 0)),
        ],
        out_specs=[
            pl.BlockSpec((FBLK, H), lambda i: (i, 0)),
            pl.BlockSpec((B, F), lambda i: (0, 0)),
        ],
        out_shape=[
            jax.ShapeDtypeStruct((F, H), jnp.float32),
            jax.ShapeDtypeStruct((B, F), jnp.int32),
        ],
    )(pe, w_pos, bsum, align_phone)


def _sc_body(enc_hbm, gidx_hbm, pitch_hbm, beats_hbm, posd_hbm, wp_hbm, wb_hbm,
             out_hbm, idx_v, rows_v, posd_v, pitch_v, beats_v, wp_v, wb_v, sem):
    wid = lax.axis_index("s") * NC + lax.axis_index("c")
    base = wid * FB
    pltpu.sync_copy(posd_hbm.at[pl.ds(base, FB), :], posd_v)
    pltpu.sync_copy(wp_hbm, wp_v)
    pltpu.sync_copy(wb_hbm, wb_v)

    def per_b(b, carry):
        pltpu.sync_copy(gidx_hbm.at[b, pl.ds(base, FB)], idx_v)
        pltpu.sync_copy(pitch_hbm.at[b, pl.ds(base, FB)], pitch_v)
        pltpu.sync_copy(beats_hbm.at[b, pl.ds(base, FB)], beats_v)
        pltpu.async_copy(enc_hbm.at[idx_v], rows_v, sem).wait()

        def per_g(g, c):
            pvec = pitch_v[pl.ds(g * L, L)]
            bvec = beats_v[pl.ds(g * L, L)]
            for j in range(L):
                f = g * L + j
                pf = jnp.full((L,), pvec[j], jnp.float32)
                bf = jnp.full((L,), bvec[j], jnp.float32)
                for hv in range(H // L):
                    sl = pl.ds(hv * L, L)
                    t = pf * wp_v[sl] + bf * wb_v[sl] + posd_v[f, sl]
                    plsc.addupdate(rows_v.at[f, sl], t)
            return c

        lax.fori_loop(0, FB // L, per_g, 0)
        pltpu.sync_copy(rows_v, out_hbm.at[b, pl.ds(base, FB), :])
        return carry

    lax.fori_loop(0, B, per_b, 0)


@functools.lru_cache(maxsize=1)
def _sc_main():
    return pl.kernel(
        _sc_body,
        out_type=jax.ShapeDtypeStruct((B, F, H), jnp.float32),
        mesh=plsc.VectorSubcoreMesh(
            core_axis_name="c", subcore_axis_name="s",
            num_cores=NC, num_subcores=NS,
        ),
        scratch_types=[
            pltpu.VMEM((FB,), jnp.int32),
            pltpu.VMEM((FB, H), jnp.float32),
            pltpu.VMEM((FB, H), jnp.float32),
            pltpu.VMEM((FB,), jnp.float32),
            pltpu.VMEM((FB,), jnp.float32),
            pltpu.VMEM((H,), jnp.float32),
            pltpu.VMEM((H,), jnp.float32),
            pltpu.SemaphoreType.DMA,
        ],
    )


def kernel(encoder_out, align_phone, pitch, beats, W_pitch, b_pitch, W_beats,
           b_beats, W_pos, b_pos):
    pe = jnp.asarray(_PE)
    bsum = (b_pitch + b_beats + b_pos).reshape(1, H)
    posd, gidx = _tc_prep(pe, W_pos, bsum, align_phone.astype(jnp.int32))
    enc_flat = encoder_out.reshape(B * P, H)
    return _sc_main()(
        enc_flat, gidx, pitch, beats, posd,
        W_pitch.reshape(H), W_beats.reshape(H),
    )


# speedup vs baseline: 11.8855x; 4.6353x over previous
import functools

import numpy as np
import jax
import jax.numpy as jnp
from jax import lax
from jax.experimental import pallas as pl
from jax.experimental.pallas import tpu as pltpu
from jax.experimental.pallas import tpu_sc as plsc

B, P, F, H = 16, 512, 4096, 256
NC, NS, L = 2, 16, 16
NW = NC * NS
FB = F // NW
FBLK = 512


def _pe_np():
    pos = np.arange(F, dtype=np.float32)[:, None]
    div = np.exp(np.arange(0, H, 2).astype(np.float32) * (-np.log(10000.0) / H))
    pe = np.zeros((F, H), dtype=np.float32)
    pe[:, 0::2] = np.sin(pos * div)
    pe[:, 1::2] = np.cos(pos * div)
    return pe


_PE = _pe_np()


def _tc_prep_body(pe_ref, wpos_ref, bsum_ref, align_ref, posd_ref, gidx_ref):
    posd_ref[...] = (
        jnp.dot(pe_ref[...], wpos_ref[...], preferred_element_type=jnp.float32)
        + bsum_ref[...]
    )

    @pl.when(pl.program_id(0) == 0)
    def _():
        a = align_ref[...]
        prev = jnp.concatenate([jnp.zeros((B, 1), a.dtype), a[:, :-1]], axis=1)
        x = (a != prev).astype(jnp.int32)
        k = 1
        while k < F:
            shifted = jnp.concatenate(
                [jnp.zeros((B, k), jnp.int32), x[:, : F - k]], axis=1
            )
            x = x + shifted
            k *= 2
        idx = jnp.clip(x, 0, P - 1)
        b_iota = lax.broadcasted_iota(jnp.int32, (B, F), 0)
        gidx_ref[...] = idx + b_iota * P


def _tc_prep(pe, w_pos, bsum, align_phone):
    return pl.pallas_call(
        _tc_prep_body,
        grid=(F // FBLK,),
        in_specs=[
            pl.BlockSpec((FBLK, H), lambda i: (i, 0)),
            pl.BlockSpec((H, H), lambda i: (0, 0)),
            pl.BlockSpec((1, H), lambda i: (0, 0)),
            pl.BlockSpec((B, F), lambda i: (0, 0)),
        ],
        out_specs=[
            pl.BlockSpec((FBLK, H), lambda i: (i, 0)),
            pl.BlockSpec((B, F), lambda i: (0, 0)),
        ],
        out_shape=[
            jax.ShapeDtypeStruct((F, H), jnp.float32),
            jax.ShapeDtypeStruct((B, F), jnp.int32),
        ],
    )(pe, w_pos, bsum, align_phone)


def _sc_body(enc_hbm, gidx_hbm, pitch_hbm, beats_hbm, posd_hbm, wp_hbm, wb_hbm,
             out_hbm, idx_all, pa, ba, posd_v, w_v, rows0, rows1,
             gsem0, gsem1, ssem0, ssem1):
    wid = lax.axis_index("s") * NC + lax.axis_index("c")
    base = wid * FB
    pltpu.sync_copy(posd_hbm.at[pl.ds(base, FB), :], posd_v)
    pltpu.sync_copy(wp_hbm, w_v.at[0])
    pltpu.sync_copy(wb_hbm, w_v.at[1])
    pltpu.sync_copy(gidx_hbm.at[:, pl.ds(base, FB)], idx_all)
    pltpu.sync_copy(pitch_hbm.at[:, pl.ds(base, FB)], pa)
    pltpu.sync_copy(beats_hbm.at[:, pl.ds(base, FB)], ba)

    wp_c = [w_v[0, pl.ds(hv * L, L)] for hv in range(H // L)]
    wb_c = [w_v[1, pl.ds(hv * L, L)] for hv in range(H // L)]

    def compute(b, rows_v):
        def per_g(g, c):
            pvec = pa[b, pl.ds(g * L, L)]
            bvec = ba[b, pl.ds(g * L, L)]
            for j in range(L):
                f = g * L + j
                pf = jnp.full((L,), pvec[j], jnp.float32)
                bf = jnp.full((L,), bvec[j], jnp.float32)
                for hv in range(H // L):
                    sl = pl.ds(hv * L, L)
                    t = pf * wp_c[hv] + bf * wb_c[hv] + posd_v[f, sl]
                    plsc.addupdate(rows_v.at[f, sl], t)
            return c

        lax.fori_loop(0, FB // L, per_g, 0)

    def out_at(b):
        return out_hbm.at[b, pl.ds(base, FB), :]

    pltpu.async_copy(enc_hbm.at[idx_all.at[0]], rows0, gsem0)

    def body(i, c):
        b0 = 2 * i
        b1 = 2 * i + 1

        @pl.when(i > 0)
        def _():
            pltpu.make_async_copy(rows1, out_at(0), ssem1).wait()

        pltpu.async_copy(enc_hbm.at[idx_all.at[b1]], rows1, gsem1)
        pltpu.make_async_copy(enc_hbm.at[idx_all.at[b0]], rows0, gsem0).wait()
        compute(b0, rows0)
        pltpu.async_copy(rows0, out_at(b0), ssem0)
        pltpu.make_async_copy(enc_hbm.at[idx_all.at[b1]], rows1, gsem1).wait()
        compute(b1, rows1)
        pltpu.make_async_copy(rows0, out_at(0), ssem0).wait()

        @pl.when(i < B // 2 - 1)
        def _():
            pltpu.async_copy(enc_hbm.at[idx_all.at[b0 + 2]], rows0, gsem0)

        pltpu.async_copy(rows1, out_at(b1), ssem1)
        return c

    lax.fori_loop(0, B // 2, body, 0)
    pltpu.make_async_copy(rows1, out_at(0), ssem1).wait()


@functools.lru_cache(maxsize=1)
def _sc_main():
    return pl.kernel(
        _sc_body,
        out_type=jax.ShapeDtypeStruct((B, F, H), jnp.float32),
        mesh=plsc.VectorSubcoreMesh(
            core_axis_name="c", subcore_axis_name="s",
            num_cores=NC, num_subcores=NS,
        ),
        scratch_types=[
            pltpu.VMEM((B, FB), jnp.int32),
            pltpu.VMEM((B, FB), jnp.float32),
            pltpu.VMEM((B, FB), jnp.float32),
            pltpu.VMEM((FB, H), jnp.float32),
            pltpu.VMEM((2, H), jnp.float32),
            pltpu.VMEM((FB, H), jnp.float32),
            pltpu.VMEM((FB, H), jnp.float32),
            pltpu.SemaphoreType.DMA,
            pltpu.SemaphoreType.DMA,
            pltpu.SemaphoreType.DMA,
            pltpu.SemaphoreType.DMA,
        ],
    )


def kernel(encoder_out, align_phone, pitch, beats, W_pitch, b_pitch, W_beats,
           b_beats, W_pos, b_pos):
    pe = jnp.asarray(_PE)
    bsum = (b_pitch + b_beats + b_pos).reshape(1, H)
    posd, gidx = _tc_prep(pe, W_pos, bsum, align_phone.astype(jnp.int32))
    enc_flat = encoder_out.reshape(B * P, H)
    return _sc_main()(
        enc_flat, gidx, pitch, beats, posd,
        W_pitch.reshape(H), W_beats.reshape(H),
    )


# speedup vs baseline: 12.4983x; 1.0516x over previous
import functools

import numpy as np
import jax
import jax.numpy as jnp
from jax import lax
from jax.experimental import pallas as pl
from jax.experimental.pallas import tpu as pltpu
from jax.experimental.pallas import tpu_sc as plsc

B, P, F, H = 16, 512, 4096, 256
NC, NS, L = 2, 16, 16
NW = NC * NS
FB = F // NW
FBLK = 512


def _pe_np():
    pos = np.arange(F, dtype=np.float32)[:, None]
    div = np.exp(np.arange(0, H, 2).astype(np.float32) * (-np.log(10000.0) / H))
    pe = np.zeros((F, H), dtype=np.float32)
    pe[:, 0::2] = np.sin(pos * div)
    pe[:, 1::2] = np.cos(pos * div)
    return pe


_PE = _pe_np()


def _tc_prep_body(pe_ref, wpos_ref, bsum_ref, align_ref, posd_ref, gidx_ref):
    posd_ref[...] = (
        jnp.dot(pe_ref[...], wpos_ref[...], preferred_element_type=jnp.float32)
        + bsum_ref[...]
    )

    @pl.when(pl.program_id(0) == 0)
    def _():
        a = align_ref[...]
        prev = jnp.concatenate([jnp.zeros((B, 1), a.dtype), a[:, :-1]], axis=1)
        x = (a != prev).astype(jnp.int32)
        k = 1
        while k < F:
            shifted = jnp.concatenate(
                [jnp.zeros((B, k), jnp.int32), x[:, : F - k]], axis=1
            )
            x = x + shifted
            k *= 2
        idx = jnp.clip(x, 0, P - 1)
        b_iota = lax.broadcasted_iota(jnp.int32, (B, F), 0)
        gidx_ref[...] = idx + b_iota * P


def _tc_prep(pe, w_pos, bsum, align_phone):
    return pl.pallas_call(
        _tc_prep_body,
        grid=(F // FBLK,),
        in_specs=[
            pl.BlockSpec((FBLK, H), lambda i: (i, 0)),
            pl.BlockSpec((H, H), lambda i: (0, 0)),
            pl.BlockSpec((1, H), lambda i: (0, 0)),
            pl.BlockSpec((B, F), lambda i: (0, 0)),
        ],
        out_specs=[
            pl.BlockSpec((FBLK, H), lambda i: (i, 0)),
            pl.BlockSpec((B, F), lambda i: (0, 0)),
        ],
        out_shape=[
            jax.ShapeDtypeStruct((F, H), jnp.float32),
            jax.ShapeDtypeStruct((B, F), jnp.int32),
        ],
    )(pe, w_pos, bsum, align_phone)


def _sc_body(enc_hbm, gidx_hbm, pitch_hbm, beats_hbm, posd_hbm, wp_hbm, wb_hbm,
             out_hbm, idx_all, pa, ba, posd_v, w_v,
             rows0, rows1, rows2, rows3,
             gsem0, gsem1, gsem2, gsem3, ssem0, ssem1, ssem2, ssem3):
    wid = lax.axis_index("s") * NC + lax.axis_index("c")
    base = wid * FB
    pltpu.sync_copy(posd_hbm.at[pl.ds(base, FB), :], posd_v)
    pltpu.sync_copy(wp_hbm, w_v.at[0])
    pltpu.sync_copy(wb_hbm, w_v.at[1])
    pltpu.sync_copy(gidx_hbm.at[:, pl.ds(base, FB)], idx_all)
    pltpu.sync_copy(pitch_hbm.at[:, pl.ds(base, FB)], pa)
    pltpu.sync_copy(beats_hbm.at[:, pl.ds(base, FB)], ba)

    wp_c = [w_v[0, pl.ds(hv * L, L)] for hv in range(H // L)]
    wb_c = [w_v[1, pl.ds(hv * L, L)] for hv in range(H // L)]

    CF = FB // 2
    NCHUNK = 2 * B
    rings = (rows0, rows1, rows2, rows3)
    gsems = (gsem0, gsem1, gsem2, gsem3)
    ssems = (ssem0, ssem1, ssem2, ssem3)

    def idx_at(i, k):
        return idx_all.at[2 * i + k // 2, pl.ds((k % 2) * CF, CF)]

    def out_at(i, k):
        return out_hbm.at[2 * i + k // 2, pl.ds(base + (k % 2) * CF, CF), :]

    def compute(i, k, rows_v):
        b = 2 * i + k // 2
        off = (k % 2) * CF

        def per_g(g, c):
            pvec = pa[b, pl.ds(off + g * L, L)]
            bvec = ba[b, pl.ds(off + g * L, L)]
            for j in range(L):
                fl = off + g * L + j
                fr = g * L + j
                pf = jnp.full((L,), pvec[j], jnp.float32)
                bf = jnp.full((L,), bvec[j], jnp.float32)
                for hv in range(H // L):
                    sl = pl.ds(hv * L, L)
                    t = pf * wp_c[hv] + bf * wb_c[hv] + posd_v[fl, sl]
                    plsc.addupdate(rows_v.at[fr, sl], t)
            return c

        lax.fori_loop(0, CF // L, per_g, 0)

    for k in range(3):
        pltpu.async_copy(enc_hbm.at[idx_at(0, k)], rings[k], gsems[k])

    def body(i, c):
        for k in range(4):
            pltpu.make_async_copy(enc_hbm.at[idx_at(i, k)], rings[k],
                                  gsems[k]).wait()
            compute(i, k, rings[k])
            pltpu.async_copy(rings[k], out_at(i, k), ssems[k])
            k3 = (k + 3) % 4
            i3 = i + (k + 3) // 4
            if k == 0:
                @pl.when(i > 0)
                def _():
                    pltpu.make_async_copy(rings[k3], out_at(0, k3),
                                          ssems[k3]).wait()
                pltpu.async_copy(enc_hbm.at[idx_at(i3, k3)], rings[k3],
                                 gsems[k3])
            else:
                last_i = (NCHUNK - 4 - k) // 4
                @pl.when(i <= last_i)
                def _():
                    pltpu.make_async_copy(rings[k3], out_at(0, k3),
                                          ssems[k3]).wait()
                    pltpu.async_copy(enc_hbm.at[idx_at(i3, k3)], rings[k3],
                                     gsems[k3])
        return c

    lax.fori_loop(0, NCHUNK // 4, body, 0)
    for k in range(4):
        pltpu.make_async_copy(rings[k], out_at(0, k), ssems[k]).wait()


@functools.lru_cache(maxsize=1)
def _sc_main():
    return pl.kernel(
        _sc_body,
        out_type=jax.ShapeDtypeStruct((B, F, H), jnp.float32),
        mesh=plsc.VectorSubcoreMesh(
            core_axis_name="c", subcore_axis_name="s",
            num_cores=NC, num_subcores=NS,
        ),
        scratch_types=[
            pltpu.VMEM((B, FB), jnp.int32),
            pltpu.VMEM((B, FB), jnp.float32),
            pltpu.VMEM((B, FB), jnp.float32),
            pltpu.VMEM((FB, H), jnp.float32),
            pltpu.VMEM((2, H), jnp.float32),
            pltpu.VMEM((FB // 2, H), jnp.float32),
            pltpu.VMEM((FB // 2, H), jnp.float32),
            pltpu.VMEM((FB // 2, H), jnp.float32),
            pltpu.VMEM((FB // 2, H), jnp.float32),
            pltpu.SemaphoreType.DMA,
            pltpu.SemaphoreType.DMA,
            pltpu.SemaphoreType.DMA,
            pltpu.SemaphoreType.DMA,
            pltpu.SemaphoreType.DMA,
            pltpu.SemaphoreType.DMA,
            pltpu.SemaphoreType.DMA,
            pltpu.SemaphoreType.DMA,
        ],
    )


def kernel(encoder_out, align_phone, pitch, beats, W_pitch, b_pitch, W_beats,
           b_beats, W_pos, b_pos):
    pe = jnp.asarray(_PE)
    bsum = (b_pitch + b_beats + b_pos).reshape(1, H)
    posd, gidx = _tc_prep(pe, W_pos, bsum, align_phone.astype(jnp.int32))
    enc_flat = encoder_out.reshape(B * P, H)
    return _sc_main()(
        enc_flat, gidx, pitch, beats, posd,
        W_pitch.reshape(H), W_beats.reshape(H),
    )
